# Initial kernel scaffold; baseline (speedup 1.0000x reference)
import jax
import jax.numpy as jnp
from jax.experimental import pallas as pl


def kernel(inp, w0, w1, w2, fs0, fs2):
    raise NotImplementedError("write your pallas kernel here")


# speedup vs baseline: 1.7875x; 1.7875x over previous
import functools

import jax
import jax.numpy as jnp
from jax import lax
from jax.experimental import pallas as pl
from jax.experimental.pallas import tpu as pltpu
from jax.experimental.pallas import tpu_sc as plsc

F = 768
S = 2048
B = 2
N = B * S
E = 8
C3 = 2304
TE = N // E
NC = 2
NS = 16
NW = NC * NS


def _sc_row_gather(n_out, n_tab, width, chunk):
    per_w = n_out // NW
    n_chunks = per_w // chunk
    assert per_w % chunk == 0 and chunk <= 128 and chunk % 8 == 0
    mesh = plsc.VectorSubcoreMesh(core_axis_name="c", subcore_axis_name="s")

    @functools.partial(
        pl.kernel,
        out_type=jax.ShapeDtypeStruct((n_out, width), jnp.float32),
        mesh=mesh,
        scratch_types=[
            pltpu.VMEM((chunk,), jnp.int32),
            pltpu.VMEM((chunk, width), jnp.float32),
            pltpu.SemaphoreType.DMA,
        ],
    )
    def k(table_hbm, idx_hbm, out_hbm, idx_v, rows_v, sem):
        wid = lax.axis_index("s") * NC + lax.axis_index("c")
        base = wid * per_w
        for i in range(n_chunks):
            off = base + i * chunk
            pltpu.sync_copy(idx_hbm.at[pl.ds(off, chunk)], idx_v)
            pltpu.async_copy(table_hbm.at[idx_v], rows_v, sem).wait()
            pltpu.sync_copy(rows_v, out_hbm.at[pl.ds(off, chunk)])

    return k


def _transpose_to_tokens(x):
    Sb = 256

    def body(x_ref, o_ref):
        o_ref[...] = x_ref[0].T

    return pl.pallas_call(
        body,
        grid=(B, S // Sb),
        in_specs=[pl.BlockSpec((1, F, Sb), lambda b, s: (b, 0, s))],
        out_specs=pl.BlockSpec((Sb, F), lambda b, s: (b * (S // Sb) + s, 0)),
        out_shape=jax.ShapeDtypeStruct((N, F), jnp.float32),
    )(x)


def _transpose_to_features(x):
    Sb = 256

    def body(x_ref, o_ref):
        o_ref[...] = x_ref[...].T[None]

    return pl.pallas_call(
        body,
        grid=(B, S // Sb),
        in_specs=[pl.BlockSpec((Sb, F), lambda b, s: (b * (S // Sb) + s, 0))],
        out_specs=pl.BlockSpec((1, F, Sb), lambda b, s: (b, 0, s)),
        out_shape=jax.ShapeDtypeStruct((B, F, S), jnp.float32),
    )(x)


def _moe0_matmul(xs, w0):
    def body(x_ref, w_ref, o_ref):
        for g in range(4):
            xg = x_ref[:, g * 192:(g + 1) * 192]
            o_ref[:, g * 576:(g + 1) * 576] = jnp.dot(
                xg, w_ref[g], preferred_element_type=jnp.float32,
                precision=lax.Precision.HIGHEST)

    return pl.pallas_call(
        body,
        grid=(E,),
        in_specs=[pl.BlockSpec((TE, F), lambda e: (e, 0)),
                  pl.BlockSpec((4, 192, 576), lambda e: (e, 0, 0))],
        out_specs=pl.BlockSpec((TE, C3), lambda e: (e, 0)),
        out_shape=jax.ShapeDtypeStruct((N, C3), jnp.float32),
    )(xs, w0)


def _norm_block(s0, s1, shift):
    s0r = jnp.maximum(s0, 0.0)
    x = s0r * s0r * s0r * s1 + shift
    x = x - jnp.mean(x, axis=1, keepdims=True)
    ssq = jnp.sum(x * x, axis=1, keepdims=True)
    return x * lax.rsqrt(ssq * (1.0 / F))


def _cum_norm(d_tok):
    Rb = 256
    per_batch = S // Rb

    def body(dep_ref, sc_ref, sh_ref, o_ref, carry_ref):
        i = pl.program_id(0)

        @pl.when(i % per_batch == 0)
        def _():
            carry_ref[...] = jnp.zeros_like(carry_ref)

        r = lax.broadcasted_iota(jnp.int32, (Rb, Rb), 0)
        c = lax.broadcasted_iota(jnp.int32, (Rb, Rb), 1)
        ltri = (r >= c).astype(jnp.float32)
        cum = jnp.dot(ltri, dep_ref[...], preferred_element_type=jnp.float32,
                      precision=lax.Precision.HIGHEST) + carry_ref[...]
        carry_ref[...] = cum[Rb - 1:Rb, :]
        srow = (i % per_batch) * Rb + lax.broadcasted_iota(
            jnp.int32, (Rb, 1), 0)
        s0 = cum / (srow + 1).astype(jnp.float32)
        o_ref[...] = _norm_block(s0, sc_ref[...], sh_ref[...])

    return pl.pallas_call(
        body,
        grid=(N // Rb,),
        in_specs=[pl.BlockSpec((Rb, F), lambda i: (i, 0)),
                  pl.BlockSpec((Rb, F), lambda i: (i, 1)),
                  pl.BlockSpec((Rb, F), lambda i: (i, 2))],
        out_specs=pl.BlockSpec((Rb, F), lambda i: (i, 0)),
        out_shape=jax.ShapeDtypeStruct((N, F), jnp.float32),
        scratch_shapes=[pltpu.VMEM((1, F), jnp.float32)],
    )(d_tok, d_tok, d_tok)


def _conv_norm(x1, w1t):
    Rb = 512
    per_batch = S // Rb

    def body(cur_ref, prev_ref, w_ref, o_ref):
        i = pl.program_id(0)
        cur = cur_ref[...]
        tail = prev_ref[Rb - 2:, :]
        tail = jnp.where(i % per_batch == 0, 0.0, tail)
        ext = jnp.concatenate([tail, cur], axis=0)
        shifted = [ext[0:Rb], ext[1:Rb + 1], cur]
        cols = []
        for g in range(4):
            acc = None
            for k in range(3):
                xg = shifted[k][:, g * 192:(g + 1) * 192]
                wgk = w_ref[k, g * 576:(g + 1) * 576, :]
                pk = lax.dot_general(
                    xg, wgk, (((1,), (1,)), ((), ())),
                    preferred_element_type=jnp.float32,
                    precision=lax.Precision.HIGHEST)
                acc = pk if acc is None else acc + pk
            cols.append(acc)
        conv = jnp.concatenate(cols, axis=1)
        o_ref[...] = _norm_block(conv[:, :768], conv[:, 768:1536],
                                 conv[:, 1536:])

    return pl.pallas_call(
        body,
        grid=(N // Rb,),
        in_specs=[
            pl.BlockSpec((Rb, F), lambda i: (i, 0)),
            pl.BlockSpec((Rb, F), lambda i: (jnp.maximum(i - 1, 0), 0)),
            pl.BlockSpec((3, C3, 192), lambda i: (0, 0, 0)),
        ],
        out_specs=pl.BlockSpec((Rb, F), lambda i: (i, 0)),
        out_shape=jax.ShapeDtypeStruct((N, F), jnp.float32),
    )(x1, x1, w1t)


def _moe2_matmul(xs2, wsm, gcol):
    def body(x_ref, w_ref, g_ref, o_ref):
        x = x_ref[...]
        gc = g_ref[...]
        for g in range(4):
            xg = x * (gc == g).astype(jnp.float32)
            o_ref[:, g * 192:(g + 1) * 192] = jnp.dot(
                xg, w_ref[0, :, :192], preferred_element_type=jnp.float32,
                precision=lax.Precision.HIGHEST)

    return pl.pallas_call(
        body,
        grid=(E,),
        in_specs=[pl.BlockSpec((TE, F), lambda e: (e, 0)),
                  pl.BlockSpec((1, F, 256), lambda e: (e, 0, 0)),
                  pl.BlockSpec((1, F), lambda e: (0, 0))],
        out_specs=pl.BlockSpec((TE, F), lambda e: (e, 0)),
        out_shape=jax.ShapeDtypeStruct((N, F), jnp.float32),
    )(xs2, wsm, gcol)


def _routing_indices():
    rkey = jax.random.key(1234)
    ka, kb = jax.random.split(rkey)
    idxs = []
    for key in (ka, kb):
        perm = jax.random.permutation(key, N).astype(jnp.int32)
        j = jnp.arange(N, dtype=jnp.int32)
        t_of_j = (j % TE) * E + j // TE
        src = perm[t_of_j]
        oslot = (perm % E) * TE + perm // E
        idxs.append((src, oslot))
    return idxs


def kernel(inp, w0, w1, w2, fs0, fs2):
    (src0, oslot0), (src2, oslot2) = _routing_indices()
    fs2_inv = jnp.argsort(fs2).astype(jnp.int32)
    idx_shuf0 = (jnp.repeat(jnp.arange(B, dtype=jnp.int32) * F, F)
                 + jnp.tile(fs0.astype(jnp.int32), B))
    qidx = (jnp.arange(E, dtype=jnp.int32)[:, None] * F
            + fs2_inv[None, :]).reshape(-1)
    gcol = (fs2_inv // 192).reshape(1, F)
    w1t = jnp.transpose(w1, (2, 0, 1))

    shuf = _sc_row_gather(B * F, B * F, S, 48)(inp.reshape(B * F, S),
                                               idx_shuf0)
    xtm = _transpose_to_tokens(shuf.reshape(B, F, S))
    xs0 = _sc_row_gather(N, N, F, 128)(xtm, src0)
    y0 = _moe0_matmul(xs0, w0)
    d_tok = _sc_row_gather(N, N, C3, 32)(y0, oslot0)
    x1 = _cum_norm(d_tok)
    x2 = _conv_norm(x1, w1t)
    w2p = jnp.pad(w2.reshape(E * F, 192), ((0, 0), (0, 64)))
    wsm = _sc_row_gather(E * F, E * F, 256, 96)(w2p, qidx)
    xs2 = _sc_row_gather(N, N, F, 128)(x2, src2)
    y2 = _moe2_matmul(xs2, wsm.reshape(E, F, 256), gcol)
    out_tok = _sc_row_gather(N, N, F, 128)(y2, oslot2)
    return _transpose_to_features(out_tok)


# speedup vs baseline: 2.9554x; 1.6534x over previous
import functools

import jax
import jax.numpy as jnp
from jax import lax
from jax.experimental import pallas as pl
from jax.experimental.pallas import tpu as pltpu
from jax.experimental.pallas import tpu_sc as plsc

F = 768
S = 2048
B = 2
N = B * S
E = 8
C3 = 2304
TE = N // E
NC = 2
NS = 16
NW = NC * NS


def _sc_row_gather(n_out, n_tab, width, chunk):
    per_w = n_out // NW
    n_chunks = per_w // chunk
    assert per_w % chunk == 0 and chunk <= 128 and chunk % 8 == 0
    mesh = plsc.VectorSubcoreMesh(core_axis_name="c", subcore_axis_name="s")

    @functools.partial(
        pl.kernel,
        out_type=jax.ShapeDtypeStruct((n_out, width), jnp.float32),
        mesh=mesh,
        scratch_types=[
            pltpu.VMEM((chunk,), jnp.int32),
            pltpu.VMEM((chunk, width), jnp.float32),
            pltpu.SemaphoreType.DMA,
        ],
    )
    def k(table_hbm, idx_hbm, out_hbm, idx_v, rows_v, sem):
        wid = lax.axis_index("s") * NC + lax.axis_index("c")
        base = wid * per_w
        for i in range(n_chunks):
            off = base + i * chunk
            pltpu.sync_copy(idx_hbm.at[pl.ds(off, chunk)], idx_v)
            pltpu.async_copy(table_hbm.at[idx_v], rows_v, sem).wait()
            pltpu.sync_copy(rows_v, out_hbm.at[pl.ds(off, chunk)])

    return k


def _transpose_to_tokens(x):
    Sb = 256

    def body(x_ref, o_ref):
        o_ref[...] = x_ref[0].T

    return pl.pallas_call(
        body,
        grid=(B, S // Sb),
        in_specs=[pl.BlockSpec((1, F, Sb), lambda b, s: (b, 0, s))],
        out_specs=pl.BlockSpec((Sb, F), lambda b, s: (b * (S // Sb) + s, 0)),
        out_shape=jax.ShapeDtypeStruct((N, F), jnp.float32),
    )(x)


def _transpose_to_features(x):
    Sb = 256

    def body(x_ref, o_ref):
        o_ref[...] = x_ref[...].T[None]

    return pl.pallas_call(
        body,
        grid=(B, S // Sb),
        in_specs=[pl.BlockSpec((Sb, F), lambda b, s: (b * (S // Sb) + s, 0))],
        out_specs=pl.BlockSpec((1, F, Sb), lambda b, s: (b, 0, s)),
        out_shape=jax.ShapeDtypeStruct((B, F, S), jnp.float32),
    )(x)


def _moe0_matmul(xs, w0):
    def body(x_ref, w_ref, o_ref):
        for g in range(4):
            xg = x_ref[:, g * 192:(g + 1) * 192]
            o_ref[:, g * 576:(g + 1) * 576] = jnp.dot(
                xg, w_ref[g], preferred_element_type=jnp.float32,
                precision=lax.Precision.DEFAULT)

    return pl.pallas_call(
        body,
        grid=(E,),
        in_specs=[pl.BlockSpec((TE, F), lambda e: (e, 0)),
                  pl.BlockSpec((4, 192, 576), lambda e: (e, 0, 0))],
        out_specs=pl.BlockSpec((TE, C3), lambda e: (e, 0)),
        out_shape=jax.ShapeDtypeStruct((N, C3), jnp.float32),
    )(xs, w0)


def _norm_block(s0, s1, shift):
    s0r = jnp.maximum(s0, 0.0)
    x = s0r * s0r * s0r * s1 + shift
    x = x - jnp.mean(x, axis=1, keepdims=True)
    ssq = jnp.sum(x * x, axis=1, keepdims=True)
    return x * lax.rsqrt(ssq * (1.0 / F))


def _cum_norm_conv_norm(d_tok, w1t):
    Rb = 256
    per_batch = S // Rb

    def body(dep_ref, sc_ref, sh_ref, w_ref, o_ref, carry_ref, tail_ref):
        i = pl.program_id(0)

        @pl.when(i % per_batch == 0)
        def _():
            carry_ref[...] = jnp.zeros_like(carry_ref)
            tail_ref[...] = jnp.zeros_like(tail_ref)

        r = lax.broadcasted_iota(jnp.int32, (Rb, Rb), 0)
        c = lax.broadcasted_iota(jnp.int32, (Rb, Rb), 1)
        ltri = (r >= c).astype(jnp.float32)
        cum = jnp.dot(ltri, dep_ref[...], preferred_element_type=jnp.float32,
                      precision=lax.Precision.HIGHEST) + carry_ref[...]
        carry_ref[...] = cum[Rb - 1:Rb, :]
        srow = (i % per_batch) * Rb + lax.broadcasted_iota(
            jnp.int32, (Rb, 1), 0)
        s0 = cum / (srow + 1).astype(jnp.float32)
        x1 = _norm_block(s0, sc_ref[...], sh_ref[...])

        ext = jnp.concatenate([tail_ref[...], x1], axis=0)
        tail_ref[...] = x1[Rb - 2:, :]
        shifted = [ext[0:Rb], ext[1:Rb + 1], x1]
        cols = []
        for g in range(4):
            acc = None
            for k in range(3):
                xg = shifted[k][:, g * 192:(g + 1) * 192]
                wgk = w_ref[k, g * 576:(g + 1) * 576, :]
                pk = lax.dot_general(
                    xg, wgk, (((1,), (1,)), ((), ())),
                    preferred_element_type=jnp.float32,
                    precision=lax.Precision.DEFAULT)
                acc = pk if acc is None else acc + pk
            cols.append(acc)
        conv = jnp.concatenate(cols, axis=1)
        o_ref[...] = _norm_block(conv[:, :768], conv[:, 768:1536],
                                 conv[:, 1536:])

    return pl.pallas_call(
        body,
        grid=(N // Rb,),
        in_specs=[pl.BlockSpec((Rb, F), lambda i: (i, 0)),
                  pl.BlockSpec((Rb, F), lambda i: (i, 1)),
                  pl.BlockSpec((Rb, F), lambda i: (i, 2)),
                  pl.BlockSpec((3, C3, 192), lambda i: (0, 0, 0))],
        out_specs=pl.BlockSpec((Rb, F), lambda i: (i, 0)),
        out_shape=jax.ShapeDtypeStruct((N, F), jnp.float32),
        scratch_shapes=[pltpu.VMEM((1, F), jnp.float32),
                        pltpu.VMEM((2, F), jnp.float32)],
    )(d_tok, d_tok, d_tok, w1t)


def _moe2_matmul(xs2, wsm, gcol):
    def body(x_ref, w_ref, g_ref, o_ref):
        x = x_ref[...]
        gc = g_ref[...]
        for g in range(4):
            xg = x * (gc == g).astype(jnp.float32)
            o_ref[:, g * 192:(g + 1) * 192] = jnp.dot(
                xg, w_ref[0, :, :192], preferred_element_type=jnp.float32,
                precision=lax.Precision.DEFAULT)

    return pl.pallas_call(
        body,
        grid=(E,),
        in_specs=[pl.BlockSpec((TE, F), lambda e: (e, 0)),
                  pl.BlockSpec((1, F, 256), lambda e: (e, 0, 0)),
                  pl.BlockSpec((1, F), lambda e: (0, 0))],
        out_specs=pl.BlockSpec((TE, F), lambda e: (e, 0)),
        out_shape=jax.ShapeDtypeStruct((N, F), jnp.float32),
    )(xs2, wsm, gcol)


def _routing_indices():
    rkey = jax.random.key(1234)
    ka, kb = jax.random.split(rkey)
    idxs = []
    for key in (ka, kb):
        perm = jax.random.permutation(key, N).astype(jnp.int32)
        j = jnp.arange(N, dtype=jnp.int32)
        t_of_j = (j % TE) * E + j // TE
        src = perm[t_of_j]
        oslot = (perm % E) * TE + perm // E
        idxs.append((src, oslot))
    return idxs


def kernel(inp, w0, w1, w2, fs0, fs2):
    (src0, oslot0), (src2, oslot2) = _routing_indices()
    fs2_inv = jnp.argsort(fs2).astype(jnp.int32)
    idx_shuf0 = (jnp.repeat(jnp.arange(B, dtype=jnp.int32) * F, F)
                 + jnp.tile(fs0.astype(jnp.int32), B))
    qidx = (jnp.arange(E, dtype=jnp.int32)[:, None] * F
            + fs2_inv[None, :]).reshape(-1)
    gcol = (fs2_inv // 192).reshape(1, F)
    w1t = jnp.transpose(w1, (2, 0, 1))

    shuf = _sc_row_gather(B * F, B * F, S, 48)(inp.reshape(B * F, S),
                                               idx_shuf0)
    xtm = _transpose_to_tokens(shuf.reshape(B, F, S))
    xs0 = _sc_row_gather(N, N, F, 128)(xtm, src0)
    y0 = _moe0_matmul(xs0, w0)
    d_tok = _sc_row_gather(N, N, C3, 32)(y0, oslot0)
    x2 = _cum_norm_conv_norm(d_tok, w1t)
    w2p = jnp.pad(w2.reshape(E * F, 192), ((0, 0), (0, 64)))
    wsm = _sc_row_gather(E * F, E * F, 256, 96)(w2p, qidx)
    xs2 = _sc_row_gather(N, N, F, 128)(x2, src2)
    y2 = _moe2_matmul(xs2, wsm.reshape(E, F, 256), gcol)
    out_tok = _sc_row_gather(N, N, F, 128)(y2, oslot2)
    return _transpose_to_features(out_tok)


# speedup vs baseline: 3.0485x; 1.0315x over previous
import functools

import jax
import jax.numpy as jnp
from jax import lax
from jax.experimental import pallas as pl
from jax.experimental.pallas import tpu as pltpu
from jax.experimental.pallas import tpu_sc as plsc

F = 768
S = 2048
B = 2
N = B * S
E = 8
C3 = 2304
TE = N // E
NC = 2
NS = 16
NW = NC * NS


def _sc_row_gather(n_out, n_tab, width, chunk):
    per_w = n_out // NW
    n_chunks = per_w // chunk
    assert per_w % chunk == 0 and chunk <= 128 and chunk % 8 == 0
    mesh = plsc.VectorSubcoreMesh(core_axis_name="c", subcore_axis_name="s")

    @functools.partial(
        pl.kernel,
        out_type=jax.ShapeDtypeStruct((n_out, width), jnp.float32),
        mesh=mesh,
        scratch_types=[
            pltpu.VMEM((chunk,), jnp.int32),
            pltpu.VMEM((chunk, width), jnp.float32),
            pltpu.SemaphoreType.DMA,
        ],
    )
    def k(table_hbm, idx_hbm, out_hbm, idx_v, rows_v, sem):
        wid = lax.axis_index("s") * NC + lax.axis_index("c")
        base = wid * per_w
        for i in range(n_chunks):
            off = base + i * chunk
            pltpu.sync_copy(idx_hbm.at[pl.ds(off, chunk)], idx_v)
            pltpu.async_copy(table_hbm.at[idx_v], rows_v, sem).wait()
            pltpu.sync_copy(rows_v, out_hbm.at[pl.ds(off, chunk)])

    return k


def _transpose_to_tokens(x):
    Sb = 256

    def body(x_ref, o_ref):
        o_ref[...] = x_ref[0].T

    return pl.pallas_call(
        body,
        grid=(B, S // Sb),
        in_specs=[pl.BlockSpec((1, F, Sb), lambda b, s: (b, 0, s))],
        out_specs=pl.BlockSpec((Sb, F), lambda b, s: (b * (S // Sb) + s, 0)),
        out_shape=jax.ShapeDtypeStruct((N, F), jnp.float32),
    )(x)


def _transpose_to_features(x):
    Sb = 256

    def body(x_ref, o_ref):
        o_ref[...] = x_ref[...].T[None]

    return pl.pallas_call(
        body,
        grid=(B, S // Sb),
        in_specs=[pl.BlockSpec((Sb, F), lambda b, s: (b * (S // Sb) + s, 0))],
        out_specs=pl.BlockSpec((1, F, Sb), lambda b, s: (b, 0, s)),
        out_shape=jax.ShapeDtypeStruct((B, F, S), jnp.float32),
    )(x)


def _moe0_matmul(xs, w0):
    def body(x_ref, w_ref, o_ref):
        for g in range(4):
            xg = x_ref[:, g * 192:(g + 1) * 192]
            o_ref[:, g * 576:(g + 1) * 576] = jnp.dot(
                xg, w_ref[g], preferred_element_type=jnp.float32,
                precision=lax.Precision.DEFAULT)

    return pl.pallas_call(
        body,
        grid=(E,),
        in_specs=[pl.BlockSpec((TE, F), lambda e: (e, 0)),
                  pl.BlockSpec((4, 192, 576), lambda e: (e, 0, 0))],
        out_specs=pl.BlockSpec((TE, C3), lambda e: (e, 0)),
        out_shape=jax.ShapeDtypeStruct((N, C3), jnp.float32),
    )(xs, w0)


def _norm_block(s0, s1, shift):
    s0r = jnp.maximum(s0, 0.0)
    x = s0r * s0r * s0r * s1 + shift
    x = x - jnp.mean(x, axis=1, keepdims=True)
    ssq = jnp.sum(x * x, axis=1, keepdims=True)
    return x * lax.rsqrt(ssq * (1.0 / F))


def _cum_norm_conv_norm(d_tok, w1t):
    Rb = 256
    per_batch = S // Rb

    def body(dep_ref, sc_ref, sh_ref, w_ref, o_ref, carry_ref, tail_ref):
        i = pl.program_id(0)

        @pl.when(i % per_batch == 0)
        def _():
            carry_ref[...] = jnp.zeros_like(carry_ref)
            tail_ref[...] = jnp.zeros_like(tail_ref)

        r = lax.broadcasted_iota(jnp.int32, (Rb, Rb), 0)
        c = lax.broadcasted_iota(jnp.int32, (Rb, Rb), 1)
        ltri = (r >= c).astype(jnp.float32)
        cum = jnp.dot(ltri, dep_ref[...], preferred_element_type=jnp.float32,
                      precision=lax.Precision.DEFAULT) + carry_ref[...]
        carry_ref[...] = cum[Rb - 1:Rb, :]
        srow = (i % per_batch) * Rb + lax.broadcasted_iota(
            jnp.int32, (Rb, 1), 0)
        s0 = cum / (srow + 1).astype(jnp.float32)
        x1 = _norm_block(s0, sc_ref[...], sh_ref[...])

        ext = jnp.concatenate([tail_ref[...], x1], axis=0)
        tail_ref[...] = x1[Rb - 2:, :]
        shifted = [ext[0:Rb], ext[1:Rb + 1], x1]
        cols = []
        for g in range(4):
            acc = None
            for k in range(3):
                xg = shifted[k][:, g * 192:(g + 1) * 192]
                wgk = w_ref[k, g * 576:(g + 1) * 576, :]
                pk = lax.dot_general(
                    xg, wgk, (((1,), (1,)), ((), ())),
                    preferred_element_type=jnp.float32,
                    precision=lax.Precision.DEFAULT)
                acc = pk if acc is None else acc + pk
            cols.append(acc)
        conv = jnp.concatenate(cols, axis=1)
        o_ref[...] = _norm_block(conv[:, :768], conv[:, 768:1536],
                                 conv[:, 1536:])

    return pl.pallas_call(
        body,
        grid=(N // Rb,),
        in_specs=[pl.BlockSpec((Rb, F), lambda i: (i, 0)),
                  pl.BlockSpec((Rb, F), lambda i: (i, 1)),
                  pl.BlockSpec((Rb, F), lambda i: (i, 2)),
                  pl.BlockSpec((3, C3, 192), lambda i: (0, 0, 0))],
        out_specs=pl.BlockSpec((Rb, F), lambda i: (i, 0)),
        out_shape=jax.ShapeDtypeStruct((N, F), jnp.float32),
        scratch_shapes=[pltpu.VMEM((1, F), jnp.float32),
                        pltpu.VMEM((2, F), jnp.float32)],
    )(d_tok, d_tok, d_tok, w1t)


def _moe2_matmul(xs2, wsm, gcol):
    def body(x_ref, w_ref, g_ref, o_ref):
        x = x_ref[...]
        gc = g_ref[...]
        for g in range(4):
            xg = x * (gc == g).astype(jnp.float32)
            o_ref[:, g * 192:(g + 1) * 192] = jnp.dot(
                xg, w_ref[0, :, :192], preferred_element_type=jnp.float32,
                precision=lax.Precision.DEFAULT)

    return pl.pallas_call(
        body,
        grid=(E,),
        in_specs=[pl.BlockSpec((TE, F), lambda e: (e, 0)),
                  pl.BlockSpec((1, F, 256), lambda e: (e, 0, 0)),
                  pl.BlockSpec((1, F), lambda e: (0, 0))],
        out_specs=pl.BlockSpec((TE, F), lambda e: (e, 0)),
        out_shape=jax.ShapeDtypeStruct((N, F), jnp.float32),
    )(xs2, wsm, gcol)


def _routing_indices():
    rkey = jax.random.key(1234)
    ka, kb = jax.random.split(rkey)
    idxs = []
    for key in (ka, kb):
        perm = jax.random.permutation(key, N).astype(jnp.int32)
        j = jnp.arange(N, dtype=jnp.int32)
        t_of_j = (j % TE) * E + j // TE
        src = perm[t_of_j]
        oslot = (perm % E) * TE + perm // E
        idxs.append((src, oslot))
    return idxs


def kernel(inp, w0, w1, w2, fs0, fs2):
    (src0, oslot0), (src2, oslot2) = _routing_indices()
    fs2_inv = jnp.argsort(fs2).astype(jnp.int32)
    idx_shuf0 = (jnp.repeat(jnp.arange(B, dtype=jnp.int32) * F, F)
                 + jnp.tile(fs0.astype(jnp.int32), B))
    qidx = (jnp.arange(E, dtype=jnp.int32)[:, None] * F
            + fs2_inv[None, :]).reshape(-1)
    gcol = (fs2_inv // 192).reshape(1, F)
    w1t = jnp.transpose(w1, (2, 0, 1))

    shuf = _sc_row_gather(B * F, B * F, S, 48)(inp.reshape(B * F, S),
                                               idx_shuf0)
    xtm = _transpose_to_tokens(shuf.reshape(B, F, S))
    xs0 = _sc_row_gather(N, N, F, 128)(xtm, src0)
    y0 = _moe0_matmul(xs0, w0)
    d_tok = _sc_row_gather(N, N, C3, 32)(y0, oslot0)
    x2 = _cum_norm_conv_norm(d_tok, w1t)
    w2p = jnp.pad(w2.reshape(E * F, 192), ((0, 0), (0, 64)))
    wsm = _sc_row_gather(E * F, E * F, 256, 96)(w2p, qidx)
    xs2 = _sc_row_gather(N, N, F, 128)(x2, src2)
    y2 = _moe2_matmul(xs2, wsm.reshape(E, F, 256), gcol)
    out_tok = _sc_row_gather(N, N, F, 128)(y2, oslot2)
    return _transpose_to_features(out_tok)
